# speedup vs baseline: 15.4149x; 15.4149x over previous
import functools

import jax
import jax.numpy as jnp
from jax import lax
from jax.experimental import pallas as pl
from jax.experimental.pallas import tpu as pltpu
from jax.experimental.pallas import tpu_sc as plsc

NC = 2
NS = 16
NW = NC * NS
K = 80


def _deg_kernel(E, NP):
    ept = E // NW
    nit = ept // K
    sl = NP // NS
    mesh = plsc.VectorSubcoreMesh(core_axis_name="c", subcore_axis_name="s")

    @functools.partial(
        pl.kernel,
        mesh=mesh,
        out_type=jax.ShapeDtypeStruct((NC, NP), jnp.float32),
        scratch_types=[
            pltpu.VMEM((K,), jnp.int32),
            pltpu.VMEM((K,), jnp.float32),
            pltpu.VMEM_SHARED((NP,), jnp.float32),
        ],
    )
    def deg_k(row_hbm, zeros_hbm, out_hbm, idx_v, ones_v, deg_sp):
        c = lax.axis_index("c")
        s = lax.axis_index("s")
        pltpu.sync_copy(zeros_hbm, deg_sp.at[pl.ds(s * sl, sl)])
        for i in range(K // 16):
            ones_v[pl.ds(i * 16, 16)] = jnp.full((16,), 1.0, jnp.float32)
        plsc.subcore_barrier()
        tb = (c * NS + s) * ept

        def body(it, carry):
            pltpu.sync_copy(row_hbm.at[pl.ds(tb + it * K, K)], idx_v)
            pltpu.sync_copy(ones_v, deg_sp.at[idx_v], add=True)
            return carry

        lax.fori_loop(0, nit, body, 0)
        plsc.subcore_barrier()
        pltpu.sync_copy(deg_sp.at[pl.ds(s * sl, sl)],
                        out_hbm.at[c, pl.ds(s * sl, sl)])

    return deg_k


def _agg_kernel(E, N, NP, D):
    ept = E // NW
    nit = ept // K
    sl = NP // NS
    mesh = plsc.VectorSubcoreMesh(core_axis_name="c", subcore_axis_name="s")

    @functools.partial(
        pl.kernel,
        mesh=mesh,
        out_type=jax.ShapeDtypeStruct((NC, NP, D), jnp.float32),
        scratch_types=[
            pltpu.VMEM((K,), jnp.int32),
            pltpu.VMEM((K,), jnp.int32),
            pltpu.VMEM((K, D), jnp.float32),
            pltpu.VMEM_SHARED((NP, D), jnp.float32),
            pltpu.SemaphoreType.DMA,
        ],
    )
    def agg_k(row_hbm, col_hbm, xs_hbm, zeros_hbm, out_hbm,
              col_v, row_v, msg_v, acc_sp, sem):
        c = lax.axis_index("c")
        s = lax.axis_index("s")
        pltpu.sync_copy(zeros_hbm, acc_sp.at[pl.ds(s * sl, sl)])
        plsc.subcore_barrier()
        tb = (c * NS + s) * ept

        def body(it, carry):
            b = tb + it * K
            pltpu.sync_copy(col_hbm.at[pl.ds(b, K)], col_v)
            pltpu.sync_copy(row_hbm.at[pl.ds(b, K)], row_v)
            pltpu.async_copy(xs_hbm.at[col_v], msg_v, sem).wait()
            pltpu.sync_copy(msg_v, acc_sp.at[row_v], add=True)
            return carry

        lax.fori_loop(0, nit, body, 0)
        plsc.subcore_barrier()
        pltpu.sync_copy(acc_sp.at[pl.ds(s * sl, sl)],
                        out_hbm.at[c, pl.ds(s * sl, sl)])

    return agg_k


def _scale_body(dp_ref, v_ref, o_ref):
    dp = dp_ref[0]
    deg = dp[0] + dp[1]
    dinv = jnp.where(deg > 0.0, lax.rsqrt(deg), 0.0)
    v = v_ref[...]
    agg = v[0] if v.shape[0] == 1 else v[0] + v[1]
    o_ref[...] = agg * dinv[:, None]


def _scale_call(dp3, v, rb, d):
    nb = dp3.shape[0]
    p = v.shape[0]
    return pl.pallas_call(
        _scale_body,
        grid=(nb,),
        in_specs=[
            pl.BlockSpec((1, 2, rb), lambda i: (i, 0, 0)),
            pl.BlockSpec((p, rb, d), lambda i: (0, i, 0)),
        ],
        out_specs=pl.BlockSpec((rb, d), lambda i: (i, 0)),
        out_shape=jax.ShapeDtypeStruct((nb * rb, d), jnp.float32),
    )(dp3, v)


@jax.jit
def kernel(x, edge_index):
    N, D = x.shape
    E = edge_index.shape[1]
    NP = 10240
    RB = 400
    NB = N // RB
    row = edge_index[0]
    col = edge_index[1]

    zeros1 = jnp.zeros((NP // NS,), jnp.float32)
    zeros2 = jnp.zeros((NP // NS, D), jnp.float32)

    deg_parts = _deg_kernel(E, NP)(row, zeros1)
    dp3 = deg_parts[:, :N].reshape(2, NB, RB).transpose(1, 0, 2)
    xs = _scale_call(dp3, x.reshape(1, N, D), RB, D)
    parts = _agg_kernel(E, N, NP, D)(row, col, xs, zeros2)
    out = _scale_call(dp3, parts, RB, D)
    return out


# speedup vs baseline: 25.8957x; 1.6799x over previous
import functools

import jax
import jax.numpy as jnp
from jax import lax
from jax.experimental import pallas as pl
from jax.experimental.pallas import tpu as pltpu
from jax.experimental.pallas import tpu_sc as plsc

NC = 2
NS = 16
NW = NC * NS
K = 80


def _deg_kernel(E, NP):
    ept = E // NW
    nit = ept // K
    sl = NP // NS
    mesh = plsc.VectorSubcoreMesh(core_axis_name="c", subcore_axis_name="s")

    @functools.partial(
        pl.kernel,
        mesh=mesh,
        out_type=jax.ShapeDtypeStruct((NC, NP), jnp.float32),
        scratch_types=[
            pltpu.VMEM((nit, K), jnp.int32),
            pltpu.VMEM((K,), jnp.float32),
            pltpu.VMEM_SHARED((NP,), jnp.float32),
            pltpu.SemaphoreType.DMA,
        ],
    )
    def deg_k(row_hbm, zeros_hbm, out_hbm, idx_v, ones_v, deg_sp, sem):
        c = lax.axis_index("c")
        s = lax.axis_index("s")
        w = c * NS + s
        pltpu.sync_copy(row_hbm.at[w], idx_v)
        pltpu.sync_copy(zeros_hbm, deg_sp.at[pl.ds(s * sl, sl)])
        for i in range(K // 16):
            ones_v[pl.ds(i * 16, 16)] = jnp.full((16,), 1.0, jnp.float32)
        plsc.subcore_barrier()

        pltpu.async_copy(ones_v, deg_sp.at[idx_v.at[0]], sem, add=True)

        def body(it, carry):
            pltpu.async_copy(ones_v, deg_sp.at[idx_v.at[it + 1]], sem, add=True)
            pltpu.make_async_copy(ones_v, deg_sp.at[idx_v.at[it]], sem).wait()
            return carry

        lax.fori_loop(0, nit - 1, body, 0)
        pltpu.make_async_copy(ones_v, deg_sp.at[idx_v.at[nit - 1]], sem).wait()
        plsc.subcore_barrier()
        pltpu.sync_copy(deg_sp.at[pl.ds(s * sl, sl)],
                        out_hbm.at[c, pl.ds(s * sl, sl)])

    return deg_k


def _agg_kernel(E, N, NP, D):
    ept = E // NW
    nit = ept // K
    sl = NP // NS
    mesh = plsc.VectorSubcoreMesh(core_axis_name="c", subcore_axis_name="s")

    assert nit % 2 == 1

    @functools.partial(
        pl.kernel,
        mesh=mesh,
        out_type=jax.ShapeDtypeStruct((NC, NP, D), jnp.float32),
        scratch_types=[
            pltpu.VMEM((K,), jnp.int32),
            pltpu.VMEM((K,), jnp.int32),
            pltpu.VMEM((K,), jnp.int32),
            pltpu.VMEM((K,), jnp.int32),
            pltpu.VMEM((K, D), jnp.float32),
            pltpu.VMEM((K, D), jnp.float32),
            pltpu.VMEM_SHARED((NP, D), jnp.float32),
            pltpu.SemaphoreType.DMA,
            pltpu.SemaphoreType.DMA,
        ],
    )
    def agg_k(row_hbm, col_hbm, xs_hbm, zeros_hbm, out_hbm,
              c0, c1, r0, r1, m0, m1, acc_sp, semA, semB):
        c = lax.axis_index("c")
        s = lax.axis_index("s")
        tb = (c * NS + s) * ept
        pltpu.sync_copy(zeros_hbm, acc_sp.at[pl.ds(s * sl, sl)])
        plsc.subcore_barrier()

        def load_idx(it, cbuf, rbuf):
            pltpu.sync_copy(col_hbm.at[pl.ds(tb + it * K, K)], cbuf)
            pltpu.sync_copy(row_hbm.at[pl.ds(tb + it * K, K)], rbuf)

        def gather(cbuf, buf, sem):
            pltpu.async_copy(xs_hbm.at[cbuf], buf, sem)

        def gwait(cbuf, buf, sem):
            pltpu.make_async_copy(xs_hbm.at[cbuf], buf, sem).wait()

        def scat(rbuf, buf):
            pltpu.sync_copy(buf, acc_sp.at[rbuf], add=True)

        load_idx(0, c0, r0)
        gather(c0, m0, semA)
        load_idx(1, c1, r1)
        gather(c1, m1, semB)

        def body(j, carry):
            a = 2 * j
            gwait(c0, m0, semA)
            scat(r0, m0)
            load_idx(a + 2, c0, r0)
            gather(c0, m0, semA)
            gwait(c1, m1, semB)
            scat(r1, m1)

            @pl.when(a + 3 < nit)
            def _():
                load_idx(a + 3, c1, r1)
                gather(c1, m1, semB)

            return carry

        lax.fori_loop(0, (nit - 1) // 2, body, 0)
        gwait(c0, m0, semA)
        scat(r0, m0)
        plsc.subcore_barrier()
        pltpu.sync_copy(acc_sp.at[pl.ds(s * sl, sl)],
                        out_hbm.at[c, pl.ds(s * sl, sl)])

    return agg_k


def _scale_body(dp_ref, v_ref, o_ref):
    dp = dp_ref[0]
    deg = dp[0] + dp[1]
    dinv = jnp.where(deg > 0.0, lax.rsqrt(deg), 0.0)
    v = v_ref[...]
    agg = v[0] if v.shape[0] == 1 else v[0] + v[1]
    o_ref[...] = agg * dinv[:, None]


def _scale_call(dp3, v, rb, d):
    nb = dp3.shape[0]
    p = v.shape[0]
    return pl.pallas_call(
        _scale_body,
        grid=(nb,),
        in_specs=[
            pl.BlockSpec((1, 2, rb), lambda i: (i, 0, 0)),
            pl.BlockSpec((p, rb, d), lambda i: (0, i, 0)),
        ],
        out_specs=pl.BlockSpec((rb, d), lambda i: (i, 0)),
        out_shape=jax.ShapeDtypeStruct((nb * rb, d), jnp.float32),
    )(dp3, v)


@jax.jit
def kernel(x, edge_index):
    N, D = x.shape
    E = edge_index.shape[1]
    NP = 10240
    RB = 400
    NB = N // RB
    ept = E // NW
    nit = ept // K
    row = edge_index[0]
    col = edge_index[1]
    row3 = row.reshape(NW, nit, K)

    zeros1 = jnp.zeros((NP // NS,), jnp.float32)
    zeros2 = jnp.zeros((NP // NS, D), jnp.float32)

    deg_parts = _deg_kernel(E, NP)(row3, zeros1)
    dp3 = deg_parts[:, :N].reshape(2, NB, RB).transpose(1, 0, 2)
    xs = _scale_call(dp3, x.reshape(1, N, D), RB, D)
    parts = _agg_kernel(E, N, NP, D)(row, col, xs, zeros2)
    out = _scale_call(dp3, parts, RB, D)
    return out


# speedup vs baseline: 31.4717x; 1.2153x over previous
import functools

import jax
import jax.numpy as jnp
from jax import lax
from jax.experimental import pallas as pl
from jax.experimental.pallas import tpu as pltpu
from jax.experimental.pallas import tpu_sc as plsc

NC = 2
NS = 16
NW = NC * NS
K = 80


def _deg_kernel(E, NP):
    ept = E // NW
    nit = ept // K
    sl = NP // NS
    mesh = plsc.VectorSubcoreMesh(core_axis_name="c", subcore_axis_name="s")

    @functools.partial(
        pl.kernel,
        mesh=mesh,
        out_type=jax.ShapeDtypeStruct((NS, NC, NP // NS), jnp.float32),
        scratch_types=[
            pltpu.VMEM((nit, K), jnp.int32),
            pltpu.VMEM((K,), jnp.float32),
            pltpu.VMEM_SHARED((NP,), jnp.float32),
            pltpu.SemaphoreType.DMA,
        ],
    )
    def deg_k(row_hbm, zeros_hbm, out_hbm, idx_v, ones_v, deg_sp, sem):
        c = lax.axis_index("c")
        s = lax.axis_index("s")
        w = c * NS + s
        pltpu.sync_copy(row_hbm.at[w], idx_v)
        pltpu.sync_copy(zeros_hbm, deg_sp.at[pl.ds(s * sl, sl)])
        for i in range(K // 16):
            ones_v[pl.ds(i * 16, 16)] = jnp.full((16,), 1.0, jnp.float32)
        plsc.subcore_barrier()

        pltpu.async_copy(ones_v, deg_sp.at[idx_v.at[0]], sem, add=True)

        def body(it, carry):
            pltpu.async_copy(ones_v, deg_sp.at[idx_v.at[it + 1]], sem, add=True)
            pltpu.make_async_copy(ones_v, deg_sp.at[idx_v.at[it]], sem).wait()
            return carry

        lax.fori_loop(0, nit - 1, body, 0)
        pltpu.make_async_copy(ones_v, deg_sp.at[idx_v.at[nit - 1]], sem).wait()
        plsc.subcore_barrier()
        pltpu.sync_copy(deg_sp.at[pl.ds(s * sl, sl)], out_hbm.at[s, c])

    return deg_k


def _agg_kernel(E, N, NP, D):
    ept = E // NW
    nit = ept // K
    sl = NP // NS
    mesh = plsc.VectorSubcoreMesh(core_axis_name="c", subcore_axis_name="s")

    assert nit % 2 == 1

    @functools.partial(
        pl.kernel,
        mesh=mesh,
        out_type=jax.ShapeDtypeStruct((NC, NP, D), jnp.float32),
        scratch_types=[
            pltpu.VMEM((2, K), jnp.int32),
            pltpu.VMEM((2, K), jnp.int32),
            pltpu.VMEM((K, D), jnp.float32),
            pltpu.VMEM((K, D), jnp.float32),
            pltpu.VMEM_SHARED((NP, D), jnp.float32),
            pltpu.SemaphoreType.DMA,
            pltpu.SemaphoreType.DMA,
        ],
    )
    def agg_k(cr_hbm, xs_hbm, zeros_hbm, out_hbm,
              cr0, cr1, m0, m1, acc_sp, semA, semB):
        c = lax.axis_index("c")
        s = lax.axis_index("s")
        w = c * NS + s
        pltpu.sync_copy(zeros_hbm, acc_sp.at[pl.ds(s * sl, sl)])
        plsc.subcore_barrier()

        def load_idx(it, crbuf):
            pltpu.sync_copy(cr_hbm.at[w, it], crbuf)

        def gather(crbuf, buf, sem):
            pltpu.async_copy(xs_hbm.at[crbuf.at[0]], buf, sem)

        def gwait(crbuf, buf, sem):
            pltpu.make_async_copy(xs_hbm.at[crbuf.at[0]], buf, sem).wait()

        def scat(crbuf, buf):
            pltpu.sync_copy(buf, acc_sp.at[crbuf.at[1]], add=True)

        load_idx(0, cr0)
        gather(cr0, m0, semA)
        load_idx(1, cr1)
        gather(cr1, m1, semB)

        def body(j, carry):
            a = 2 * j
            gwait(cr0, m0, semA)
            scat(cr0, m0)
            load_idx(a + 2, cr0)
            gather(cr0, m0, semA)
            gwait(cr1, m1, semB)
            scat(cr1, m1)

            @pl.when(a + 3 < nit)
            def _():
                load_idx(a + 3, cr1)
                gather(cr1, m1, semB)

            return carry

        lax.fori_loop(0, (nit - 1) // 2, body, 0)
        gwait(cr0, m0, semA)
        scat(cr0, m0)
        plsc.subcore_barrier()
        pltpu.sync_copy(acc_sp.at[pl.ds(s * sl, sl)],
                        out_hbm.at[c, pl.ds(s * sl, sl)])

    return agg_k


def _scale_body(dp_ref, v_ref, o_ref):
    dp = dp_ref[0]
    deg = dp[0] + dp[1]
    dinv = jnp.where(deg > 0.0, lax.rsqrt(deg), 0.0)
    v = v_ref[...]
    agg = v[0] if v.shape[0] == 1 else v[0] + v[1]
    o_ref[...] = agg * dinv[:, None]


def _scale_call(dp3, v, n_out, d):
    nb, _, rb = dp3.shape
    p = v.shape[0]
    return pl.pallas_call(
        _scale_body,
        grid=(nb,),
        in_specs=[
            pl.BlockSpec((1, 2, rb), lambda i: (i, 0, 0)),
            pl.BlockSpec((p, rb, d), lambda i: (0, i, 0)),
        ],
        out_specs=pl.BlockSpec((rb, d), lambda i: (i, 0)),
        out_shape=jax.ShapeDtypeStruct((n_out, d), jnp.float32),
    )(dp3, v)


@jax.jit
def kernel(x, edge_index):
    N, D = x.shape
    E = edge_index.shape[1]
    NP = 10240
    ept = E // NW
    nit = ept // K
    row3 = edge_index[0].reshape(NW, nit, K)
    cr4 = edge_index.reshape(2, NW, nit, K).transpose(1, 2, 0, 3)

    zeros1 = jnp.zeros((NP // NS,), jnp.float32)
    zeros2 = jnp.zeros((NP // NS, D), jnp.float32)

    dp3 = _deg_kernel(E, NP)(row3, zeros1)
    xs = _scale_call(dp3, x.reshape(1, N, D), NP, D)
    parts = _agg_kernel(E, N, NP, D)(cr4, xs, zeros2)
    out = _scale_call(dp3, parts, N, D)
    return out
